# baseline (speedup 1.0000x reference)
import functools

import jax
import jax.numpy as jnp
from jax import lax
from jax.experimental import pallas as pl
from jax.experimental.pallas import tpu as pltpu
from jax.experimental.pallas import tpu_sc as plsc

_B, _P, _F = 4096, 10, 7
_MIN_SCORE = 0.3

_INFO = plsc.get_sparse_core_info()
_NC, _NS, _L = _INFO.num_cores, _INFO.num_subcores, _INFO.num_lanes
_NW = _NC * _NS
_RW = _B // _NW
_NREC = _RW * _P


@functools.partial(
    pl.kernel,
    mesh=plsc.VectorSubcoreMesh(core_axis_name="c", subcore_axis_name="s"),
    out_type=jax.ShapeDtypeStruct((_B, _P, _F), jnp.float32),
    compiler_params=pltpu.CompilerParams(
        needs_layout_passes=False, use_tc_tiling_on_sc=False),
    scratch_types=[
        pltpu.VMEM((_RW, _P, 3), jnp.float32),
        pltpu.VMEM((_RW, _P), jnp.float32),
        pltpu.VMEM((_RW, _P, 2), jnp.float32),
        pltpu.VMEM((_RW, _P, _F), jnp.float32),
    ],
)
def _sc_assemble(idx_hbm, conf_hbm, bbox_hbm, out_hbm,
                 idx_v, conf_v, bbox_v, out_v):
    wid = lax.axis_index("s") * _NC + lax.axis_index("c")
    rows = pl.ds(wid * _RW, _RW)
    pltpu.sync_copy(idx_hbm.at[rows], idx_v)
    pltpu.sync_copy(conf_hbm.at[rows], conf_v)
    pltpu.sync_copy(bbox_hbm.at[rows], bbox_v)

    iota = lax.iota(jnp.int32, _L)
    zero = jnp.zeros((_L,), jnp.int32)
    one = jnp.full((_L,), 1, jnp.int32)
    two = jnp.full((_L,), 2, jnp.int32)
    three = jnp.full((_L,), 3, jnp.int32)
    four = jnp.full((_L,), 4, jnp.int32)
    five = jnp.full((_L,), 5, jnp.int32)
    six = jnp.full((_L,), 6, jnp.int32)

    def grp_step(g, carry):
        j = g * _L + iota
        b = j // _P
        p = j % _P
        for c, oc in ((zero, zero), (one, one), (two, two)):
            plsc.store_scatter(out_v, [b, p, oc],
                               plsc.load_gather(idx_v, [b, p, c]))
        for c, oc in ((zero, five), (one, six)):
            plsc.store_scatter(out_v, [b, p, oc],
                               plsc.load_gather(bbox_v, [b, p, c]))
        cvals = plsc.load_gather(conf_v, [b, p])
        plsc.store_scatter(out_v, [b, p, four], cvals)
        m = jnp.where(cvals > _MIN_SCORE, jnp.float32(0.0), jnp.float32(-1.0))
        plsc.store_scatter(out_v, [b, p, three], m)
        return carry

    lax.fori_loop(0, _NREC // _L, grp_step, 0)
    pltpu.sync_copy(out_v, out_hbm.at[rows])


def kernel(topk_index, topk_confs, match_bbox_preds, meta):
    del meta
    return _sc_assemble(topk_index, topk_confs, match_bbox_preds)


# speedup vs baseline: 1.4200x; 1.4200x over previous
import functools

import numpy as np
import jax
import jax.numpy as jnp
from jax import lax
from jax.experimental import pallas as pl
from jax.experimental.pallas import tpu as pltpu
from jax.experimental.pallas import tpu_sc as plsc

_B, _P, _F = 4096, 10, 7
_MIN_SCORE = 0.3

_INFO = plsc.get_sparse_core_info()
_NC, _NS, _L = _INFO.num_cores, _INFO.num_subcores, _INFO.num_lanes
_NW = _NC * _NS
_RW = _B // _NW
_CB = 16
_NCH = _RW // _CB
_NREC = _CB * _P

_NI = _NREC * 3
_NX = _NREC * 2
_NCF = _NREC


def _rc_tables(n_feat):
    j = np.arange(_NREC * n_feat, dtype=np.int32)
    return j // n_feat, j % n_feat


_IRT, _ICT = _rc_tables(3)
_XRT, _XCT = _rc_tables(2)
_CRT, _ = _rc_tables(1)
_CBT, _CPT = _CRT // _P, _CRT % _P


@functools.partial(
    pl.kernel,
    mesh=plsc.VectorSubcoreMesh(core_axis_name="c", subcore_axis_name="s"),
    out_type=jax.ShapeDtypeStruct((_B, _P, _F), jnp.float32),
    compiler_params=pltpu.CompilerParams(needs_layout_passes=False),
    scratch_types=[
        pltpu.VMEM((_NREC, 3), jnp.float32),
        pltpu.VMEM((_CB, _P), jnp.float32),
        pltpu.VMEM((_NREC, 2), jnp.float32),
        pltpu.VMEM((_NREC, _F), jnp.float32),
        pltpu.VMEM((_NI,), jnp.int32),
        pltpu.VMEM((_NI,), jnp.int32),
        pltpu.VMEM((_NX,), jnp.int32),
        pltpu.VMEM((_NX,), jnp.int32),
        pltpu.VMEM((_NCF,), jnp.int32),
        pltpu.VMEM((_NCF,), jnp.int32),
    ],
)
def _sc_assemble(idx_hbm, conf_hbm, bbox_hbm,
                 ir_hbm, ic_hbm, xr_hbm, xc_hbm, cb_hbm, cp_hbm, out_hbm,
                 idx_v, conf_v, bbox_v, out_v, ir, ic, xr, xc, cb, cp):
    wid = lax.axis_index("s") * _NC + lax.axis_index("c")
    pltpu.sync_copy(ir_hbm, ir)
    pltpu.sync_copy(ic_hbm, ic)
    pltpu.sync_copy(xr_hbm, xr)
    pltpu.sync_copy(xc_hbm, xc)
    pltpu.sync_copy(cb_hbm, cb)
    pltpu.sync_copy(cp_hbm, cp)

    five = jnp.full((_L,), 5, jnp.int32)
    three = jnp.full((_L,), 3, jnp.int32)
    four = jnp.full((_L,), 4, jnp.int32)

    def chunk_step(ch, carry):
        rows = pl.ds(wid * _RW + ch * _CB, _CB)
        pltpu.sync_copy(idx_hbm.at[rows], idx_v.reshape(_CB, _P, 3))
        pltpu.sync_copy(conf_hbm.at[rows], conf_v)
        pltpu.sync_copy(bbox_hbm.at[rows], bbox_v.reshape(_CB, _P, 2))
        for t in range(_NI // _L):
            o = pl.ds(t * _L, _L)
            r = ir[o]
            c = ic[o]
            plsc.store_scatter(out_v, [r, c],
                               plsc.load_gather(idx_v, [r, c]))
        for t in range(_NX // _L):
            o = pl.ds(t * _L, _L)
            r = xr[o]
            c = xc[o]
            plsc.store_scatter(out_v, [r, c + five],
                               plsc.load_gather(bbox_v, [r, c]))
        for t in range(_NCF // _L):
            o = pl.ds(t * _L, _L)
            b = cb[o]
            p = cp[o]
            r = b * _P + p
            cvals = plsc.load_gather(conf_v, [b, p])
            m = jnp.where(cvals > _MIN_SCORE, jnp.float32(0.0),
                          jnp.float32(-1.0))
            plsc.store_scatter(out_v, [r, four], cvals)
            plsc.store_scatter(out_v, [r, three], m)
        pltpu.sync_copy(out_v.reshape(_CB, _P, _F), out_hbm.at[rows])
        return carry

    lax.fori_loop(0, _NCH, chunk_step, 0)


def kernel(topk_index, topk_confs, match_bbox_preds, meta):
    del meta
    return _sc_assemble(
        topk_index, topk_confs, match_bbox_preds,
        jnp.asarray(_IRT), jnp.asarray(_ICT),
        jnp.asarray(_XRT), jnp.asarray(_XCT),
        jnp.asarray(_CBT), jnp.asarray(_CPT),
    )
